# baseline (speedup 1.0000x reference)
import functools

import jax
import jax.numpy as jnp
from jax import lax
from jax.experimental import pallas as pl
from jax.experimental.pallas import tpu as pltpu
from jax.experimental.pallas import tpu_sc as plsc

NA = 100000
NE = 3200000
NC, NS = 2, 16
NW = NC * NS
EPW = NE // NW
CH = 2000
NCHUNK = EPW // CH
NG = CH // 16

BLK = 5120
NBLK = NE // BLK


def _sc_body(qpad, nbr, offs, cellb, out_s, out_g0, out_g1, out_g2,
             nbr2_v, idx_v, rows_v, off2_v, sb_v, g0b_v, g1b_v, g2b_v,
             cellb_v, sem):
    outs = (out_s, out_g0, out_g1, out_g2)
    bufs = (sb_v, g0b_v, g1b_v, g2b_v)
    wid = lax.axis_index("s") * NC + lax.axis_index("c")
    pltpu.sync_copy(cellb, cellb_v)
    cvec = [cellb_v[0], cellb_v[1], cellb_v[2]]

    iota = lax.iota(jnp.int32, 16)
    row2 = iota * 2
    zero16 = iota * 0

    def chunk_body(k, carry):
        base = wid * EPW + k * CH
        pltpu.sync_copy(nbr.at[pl.ds(base, CH)], nbr2_v)

        def flat_body2(g, carry2):
            ii = plsc.load_gather(nbr2_v, [16 * g + iota, zero16])
            jj = plsc.load_gather(nbr2_v, [16 * g + iota, zero16 + 1])
            plsc.store_scatter(idx_v, [32 * g + row2], ii)
            plsc.store_scatter(idx_v, [32 * g + row2 + 1], jj)
            return carry2

        lax.fori_loop(0, NG, flat_body2, 0, unroll=False)
        cp = pltpu.async_copy(qpad.at[idx_v], rows_v, sem)
        pltpu.sync_copy(offs.at[pl.ds(base, CH)], off2_v)
        cp.wait()

        def group_body(g, carry2):
            rbase = 32 * g
            erow = 16 * g + iota
            gk = []
            s = None
            for c in range(3):
                col = zero16 + c
                qi = plsc.load_gather(rows_v, [rbase + row2, col])
                qj = plsc.load_gather(rows_v, [rbase + row2 + 1, col])
                oc = plsc.load_gather(off2_v, [erow, col])
                rc = qi - qj - oc * cvec[c]
                sq = rc * rc
                s = sq if s is None else s + sq
                gk.append(oc * rc)
            e = pl.ds(16 * g, 16)
            sb_v[e] = s
            g0b_v[e] = gk[0]
            g1b_v[e] = gk[1]
            g2b_v[e] = gk[2]
            return carry2

        lax.fori_loop(0, NG, group_body, 0, unroll=False)
        for buf, o in zip(bufs, outs):
            pltpu.sync_copy(buf, o.at[pl.ds(base, CH)])
        return carry

    lax.fori_loop(0, NCHUNK, chunk_body, 0, unroll=False)


def _sc_stage(qpad, nbr, offs, cellb):
    mesh = plsc.VectorSubcoreMesh(core_axis_name="c", subcore_axis_name="s",
                                  num_cores=NC, num_subcores=NS)
    fn = pl.kernel(
        _sc_body,
        out_type=[jax.ShapeDtypeStruct((NE,), jnp.float32)] * 4,
        mesh=mesh,
        compiler_params=pltpu.CompilerParams(needs_layout_passes=False,
                                             use_tc_tiling_on_sc=False),
        scratch_types=[
            pltpu.VMEM((CH, 2), jnp.int32),
            pltpu.VMEM((2 * CH,), jnp.int32),
            pltpu.VMEM((2 * CH, 4), jnp.float32),
            pltpu.VMEM((CH, 3), jnp.float32),
            pltpu.VMEM((CH,), jnp.float32),
            pltpu.VMEM((CH,), jnp.float32),
            pltpu.VMEM((CH,), jnp.float32),
            pltpu.VMEM((CH,), jnp.float32),
            pltpu.VMEM((3, 16), jnp.float32),
            pltpu.SemaphoreType.DMA,
        ],
    )
    return fn(qpad, nbr, offs, cellb)


def _tc_body(sv, g0, g1, g2, w1, b1, w2, cell, v4, m4, out, acc):
    pid = pl.program_id(0)

    @pl.when(pid == 0)
    def _init():
        ke = 0.5 * jnp.sum(v4[...] * v4[...] * m4[...])
        acc[0] = ke
        acc[1] = 0.0
        acc[2] = 0.0
        acc[3] = 0.0

    s = sv[...].reshape(1, BLK)
    dinv = lax.rsqrt(s + 1e-12)
    d = (s + 1e-12) * dinv
    w1v = w1[...]
    cw = w1v * w2[...]
    t = jnp.tanh(w1v * d + b1[...])
    fp = jnp.sum(cw) - jnp.sum(cw * t * t, axis=0, keepdims=True)
    coef = (fp * dinv).reshape(BLK)
    acc[1] += jnp.sum(coef * g0[...])
    acc[2] += jnp.sum(coef * g1[...])
    acc[3] += jnp.sum(coef * g2[...])

    @pl.when(pid == NBLK - 1)
    def _fin():
        c0, c1, c2 = cell[0], cell[1], cell[2]
        vol = c0 * c1 * c2
        temperature = acc[0] / (NA * 3 * 0.5)
        p_ideal = NA * temperature / vol
        scale = 1.0 / (c0 * c1)
        out[0] = p_ideal + acc[1] * scale
        out[1] = p_ideal + acc[2] * scale
        out[2] = p_ideal + acc[3] * scale


def _tc_stage(sv, g0, g1, g2, w1t, b1c, w2, cell, v4, m4):
    return pl.pallas_call(
        _tc_body,
        grid=(NBLK,),
        in_specs=[
            pl.BlockSpec((BLK,), lambda i: (i,)),
            pl.BlockSpec((BLK,), lambda i: (i,)),
            pl.BlockSpec((BLK,), lambda i: (i,)),
            pl.BlockSpec((BLK,), lambda i: (i,)),
            pl.BlockSpec((64, 1), lambda i: (0, 0)),
            pl.BlockSpec((64, 1), lambda i: (0, 0)),
            pl.BlockSpec((64, 1), lambda i: (0, 0)),
            pl.BlockSpec(memory_space=pltpu.SMEM),
            pl.BlockSpec((NA * 4 // 128, 128), lambda i: (0, 0)),
            pl.BlockSpec((NA * 4 // 128, 128), lambda i: (0, 0)),
        ],
        out_specs=pl.BlockSpec(memory_space=pltpu.SMEM),
        out_shape=jax.ShapeDtypeStruct((3,), jnp.float32),
        scratch_shapes=[pltpu.SMEM((4,), jnp.float32)],
    )(sv, g0, g1, g2, w1t, b1c, w2, cell, v4, m4)


def kernel(q, v, nbr, offsets, mass, cell, W1, b1, W2, b2):
    qpad = jnp.pad(q, ((0, 0), (0, 1)))
    cellb = jnp.broadcast_to(cell[:, None], (3, 16))

    sv, g0, g1, g2 = _sc_stage(qpad, nbr, offsets, cellb)

    w1t = W1.reshape(64, 1)
    b1c = b1.reshape(64, 1)
    v4 = jnp.pad(v, ((0, 0), (0, 1))).reshape(NA * 4 // 128, 128)
    m4 = jnp.broadcast_to(mass[:, None], (NA, 4)).reshape(NA * 4 // 128, 128)
    return _tc_stage(sv, g0, g1, g2, w1t, b1c, W2, cell, v4, m4)


# speedup vs baseline: 9.2995x; 9.2995x over previous
import functools

import jax
import jax.numpy as jnp
from jax import lax
from jax.experimental import pallas as pl
from jax.experimental.pallas import tpu as pltpu
from jax.experimental.pallas import tpu_sc as plsc

NA = 100000
NE = 3200000
NC, NS = 2, 16
NW = NC * NS
EPW = NE // NW
CH = 2000
NCHUNK = EPW // CH
NG = CH // 16

BLK = 5120
NBLK = NE // BLK


def _sc_body(qpad, nbrt, offst, cellb, out_s, out_g0, out_g1, out_g2,
             idxi_v, idxj_v, rowsi_v, rowsj_v, o0_v, o1_v, o2_v,
             sb_v, g0b_v, g1b_v, g2b_v, cellb_v, sem):
    outs = (out_s, out_g0, out_g1, out_g2)
    bufs = (sb_v, g0b_v, g1b_v, g2b_v)
    ov = (o0_v, o1_v, o2_v)
    wid = lax.axis_index("s") * NC + lax.axis_index("c")
    pltpu.sync_copy(cellb, cellb_v)
    cvec = [cellb_v[0], cellb_v[1], cellb_v[2]]

    iota = lax.iota(jnp.int32, 16)
    zero16 = iota * 0

    def chunk_body(k, carry):
        base = wid * EPW + k * CH
        pltpu.sync_copy(nbrt.at[0, pl.ds(base, CH)], idxi_v)
        pltpu.sync_copy(nbrt.at[1, pl.ds(base, CH)], idxj_v)
        cp1 = pltpu.async_copy(qpad.at[idxi_v], rowsi_v, sem)
        cp2 = pltpu.async_copy(qpad.at[idxj_v], rowsj_v, sem)
        for c in range(3):
            pltpu.sync_copy(offst.at[c, pl.ds(base, CH)], ov[c])
        cp1.wait()
        cp2.wait()

        def group_body(g, carry2):
            erow = 16 * g + iota
            e = pl.ds(16 * g, 16)
            gk = []
            s = None
            for c in range(3):
                col = zero16 + c
                qi = plsc.load_gather(rowsi_v, [erow, col])
                qj = plsc.load_gather(rowsj_v, [erow, col])
                oc = ov[c][e]
                rc = qi - qj - oc * cvec[c]
                sq = rc * rc
                s = sq if s is None else s + sq
                gk.append(oc * rc)
            sb_v[e] = s
            g0b_v[e] = gk[0]
            g1b_v[e] = gk[1]
            g2b_v[e] = gk[2]
            return carry2

        lax.fori_loop(0, NG, group_body, 0, unroll=False)
        for buf, o in zip(bufs, outs):
            pltpu.sync_copy(buf, o.at[pl.ds(base, CH)])
        return carry

    lax.fori_loop(0, NCHUNK, chunk_body, 0, unroll=False)


def _sc_stage(qpad, nbrt, offst, cellb):
    mesh = plsc.VectorSubcoreMesh(core_axis_name="c", subcore_axis_name="s",
                                  num_cores=NC, num_subcores=NS)
    fn = pl.kernel(
        _sc_body,
        out_type=[jax.ShapeDtypeStruct((NE,), jnp.float32)] * 4,
        mesh=mesh,
        compiler_params=pltpu.CompilerParams(needs_layout_passes=False,
                                             use_tc_tiling_on_sc=False),
        scratch_types=[
            pltpu.VMEM((CH,), jnp.int32),
            pltpu.VMEM((CH,), jnp.int32),
            pltpu.VMEM((CH, 4), jnp.float32),
            pltpu.VMEM((CH, 4), jnp.float32),
            pltpu.VMEM((CH,), jnp.float32),
            pltpu.VMEM((CH,), jnp.float32),
            pltpu.VMEM((CH,), jnp.float32),
            pltpu.VMEM((CH,), jnp.float32),
            pltpu.VMEM((CH,), jnp.float32),
            pltpu.VMEM((CH,), jnp.float32),
            pltpu.VMEM((CH,), jnp.float32),
            pltpu.VMEM((3, 16), jnp.float32),
            pltpu.SemaphoreType.DMA,
        ],
    )
    return fn(qpad, nbrt, offst, cellb)


def _tc_body(sv, g0, g1, g2, w1, b1, w2, cell, v4, m4, out, acc):
    pid = pl.program_id(0)

    @pl.when(pid == 0)
    def _init():
        ke = 0.5 * jnp.sum(v4[...] * v4[...] * m4[...])
        acc[0] = ke
        acc[1] = 0.0
        acc[2] = 0.0
        acc[3] = 0.0

    s = sv[...].reshape(1, BLK)
    dinv = lax.rsqrt(s + 1e-12)
    d = (s + 1e-12) * dinv
    w1v = w1[...]
    cw = w1v * w2[...]
    t = jnp.tanh(w1v * d + b1[...])
    fp = jnp.sum(cw) - jnp.sum(cw * t * t, axis=0, keepdims=True)
    coef = (fp * dinv).reshape(BLK)
    acc[1] += jnp.sum(coef * g0[...])
    acc[2] += jnp.sum(coef * g1[...])
    acc[3] += jnp.sum(coef * g2[...])

    @pl.when(pid == NBLK - 1)
    def _fin():
        c0, c1, c2 = cell[0], cell[1], cell[2]
        vol = c0 * c1 * c2
        temperature = acc[0] / (NA * 3 * 0.5)
        p_ideal = NA * temperature / vol
        scale = 1.0 / (c0 * c1)
        out[0] = p_ideal + acc[1] * scale
        out[1] = p_ideal + acc[2] * scale
        out[2] = p_ideal + acc[3] * scale


def _tc_stage(sv, g0, g1, g2, w1t, b1c, w2, cell, v4, m4):
    return pl.pallas_call(
        _tc_body,
        grid=(NBLK,),
        in_specs=[
            pl.BlockSpec((BLK,), lambda i: (i,)),
            pl.BlockSpec((BLK,), lambda i: (i,)),
            pl.BlockSpec((BLK,), lambda i: (i,)),
            pl.BlockSpec((BLK,), lambda i: (i,)),
            pl.BlockSpec((64, 1), lambda i: (0, 0)),
            pl.BlockSpec((64, 1), lambda i: (0, 0)),
            pl.BlockSpec((64, 1), lambda i: (0, 0)),
            pl.BlockSpec(memory_space=pltpu.SMEM),
            pl.BlockSpec((NA * 4 // 128, 128), lambda i: (0, 0)),
            pl.BlockSpec((NA * 4 // 128, 128), lambda i: (0, 0)),
        ],
        out_specs=pl.BlockSpec(memory_space=pltpu.SMEM),
        out_shape=jax.ShapeDtypeStruct((3,), jnp.float32),
        scratch_shapes=[pltpu.SMEM((4,), jnp.float32)],
    )(sv, g0, g1, g2, w1t, b1c, w2, cell, v4, m4)


def kernel(q, v, nbr, offsets, mass, cell, W1, b1, W2, b2):
    qpad = jnp.pad(q, ((0, 0), (0, 1)))
    cellb = jnp.broadcast_to(cell[:, None], (3, 16))

    sv, g0, g1, g2 = _sc_stage(qpad, nbr.T, offsets.T, cellb)

    w1t = W1.reshape(64, 1)
    b1c = b1.reshape(64, 1)
    v4 = jnp.pad(v, ((0, 0), (0, 1))).reshape(NA * 4 // 128, 128)
    m4 = jnp.broadcast_to(mass[:, None], (NA, 4)).reshape(NA * 4 // 128, 128)
    return _tc_stage(sv, g0, g1, g2, w1t, b1c, W2, cell, v4, m4)


# speedup vs baseline: 11.1044x; 1.1941x over previous
import functools

import jax
import jax.numpy as jnp
from jax import lax
from jax.experimental import pallas as pl
from jax.experimental.pallas import tpu as pltpu
from jax.experimental.pallas import tpu_sc as plsc

NA = 100000
NE = 3200000
NC, NS = 2, 16
NW = NC * NS
EPW = NE // NW
CH = 2000
NCHUNK = EPW // CH
NG = CH // 16

BLK = 25600
NBLK = NE // BLK
HID = 64


def _sc_body(qpad, nbrt, offst, cellb, out_s, out_g0, out_g1, out_g2,
             idxi_v, idxj_v, rowsi_v, rowsj_v, o0_v, o1_v, o2_v,
             sb_v, g0b_v, g1b_v, g2b_v, cellb_v, sem):
    outs = (out_s, out_g0, out_g1, out_g2)
    bufs = (sb_v, g0b_v, g1b_v, g2b_v)
    ov = (o0_v, o1_v, o2_v)
    wid = lax.axis_index("s") * NC + lax.axis_index("c")
    pltpu.sync_copy(cellb, cellb_v)
    cvec = [cellb_v[0], cellb_v[1], cellb_v[2]]

    iota = lax.iota(jnp.int32, 16)
    zero16 = iota * 0

    def chunk_body(k, carry):
        base = wid * EPW + k * CH
        pltpu.sync_copy(nbrt.at[0, pl.ds(base, CH)], idxi_v)
        pltpu.sync_copy(nbrt.at[1, pl.ds(base, CH)], idxj_v)
        cp1 = pltpu.async_copy(qpad.at[idxi_v], rowsi_v, sem)
        cp2 = pltpu.async_copy(qpad.at[idxj_v], rowsj_v, sem)
        for c in range(3):
            pltpu.sync_copy(offst.at[c, pl.ds(base, CH)], ov[c])
        cp1.wait()
        cp2.wait()

        def group_body(g, carry2):
            erow = 16 * g + iota
            e = pl.ds(16 * g, 16)
            gk = []
            s = None
            for c in range(3):
                col = zero16 + c
                qi = plsc.load_gather(rowsi_v, [erow, col])
                qj = plsc.load_gather(rowsj_v, [erow, col])
                oc = ov[c][e]
                rc = qi - qj - oc * cvec[c]
                sq = rc * rc
                s = sq if s is None else s + sq
                gk.append(oc * rc)
            sb_v[e] = s
            g0b_v[e] = gk[0]
            g1b_v[e] = gk[1]
            g2b_v[e] = gk[2]
            return carry2

        lax.fori_loop(0, NG, group_body, 0, unroll=False)
        for buf, o in zip(bufs, outs):
            pltpu.sync_copy(buf, o.at[pl.ds(base, CH)])
        return carry

    lax.fori_loop(0, NCHUNK, chunk_body, 0, unroll=False)


def _sc_stage(qpad, nbrt, offst, cellb):
    mesh = plsc.VectorSubcoreMesh(core_axis_name="c", subcore_axis_name="s",
                                  num_cores=NC, num_subcores=NS)
    fn = pl.kernel(
        _sc_body,
        out_type=[jax.ShapeDtypeStruct((NE,), jnp.float32)] * 4,
        mesh=mesh,
        compiler_params=pltpu.CompilerParams(needs_layout_passes=False,
                                             use_tc_tiling_on_sc=False),
        scratch_types=[
            pltpu.VMEM((CH,), jnp.int32),
            pltpu.VMEM((CH,), jnp.int32),
            pltpu.VMEM((CH, 4), jnp.float32),
            pltpu.VMEM((CH, 4), jnp.float32),
            pltpu.VMEM((CH,), jnp.float32),
            pltpu.VMEM((CH,), jnp.float32),
            pltpu.VMEM((CH,), jnp.float32),
            pltpu.VMEM((CH,), jnp.float32),
            pltpu.VMEM((CH,), jnp.float32),
            pltpu.VMEM((CH,), jnp.float32),
            pltpu.VMEM((CH,), jnp.float32),
            pltpu.VMEM((3, 16), jnp.float32),
            pltpu.SemaphoreType.DMA,
        ],
    )
    return fn(qpad, nbrt, offst, cellb)


def _tc_body(sv, g0, g1, g2, w1, b1, w2, cell, v4, m4, out, acc):
    pid = pl.program_id(0)

    @pl.when(pid == 0)
    def _init():
        ke = 0.5 * jnp.sum(v4[...] * v4[...] * m4[...])
        acc[0] = ke
        acc[1] = 0.0
        acc[2] = 0.0
        acc[3] = 0.0

    s = sv[...]
    dinv = lax.rsqrt(s + 1e-12)
    d = (s + 1e-12) * dinv
    fs = None
    c0 = None
    for h in range(HID):
        w1s = w1[h]
        cs = w1s * w2[h]
        t = jnp.tanh(d * w1s + b1[h])
        tt = cs * (t * t)
        fs = tt if fs is None else fs + tt
        c0 = cs if c0 is None else c0 + cs
    coef = (c0 - fs) * dinv
    acc[1] += jnp.sum(coef * g0[...])
    acc[2] += jnp.sum(coef * g1[...])
    acc[3] += jnp.sum(coef * g2[...])

    @pl.when(pid == NBLK - 1)
    def _fin():
        c0, c1, c2 = cell[0], cell[1], cell[2]
        vol = c0 * c1 * c2
        temperature = acc[0] / (NA * 3 * 0.5)
        p_ideal = NA * temperature / vol
        scale = 1.0 / (c0 * c1)
        out[0] = p_ideal + acc[1] * scale
        out[1] = p_ideal + acc[2] * scale
        out[2] = p_ideal + acc[3] * scale


def _tc_stage(sv, g0, g1, g2, w1t, b1c, w2, cell, v4, m4):
    return pl.pallas_call(
        _tc_body,
        grid=(NBLK,),
        in_specs=[
            pl.BlockSpec((BLK,), lambda i: (i,)),
            pl.BlockSpec((BLK,), lambda i: (i,)),
            pl.BlockSpec((BLK,), lambda i: (i,)),
            pl.BlockSpec((BLK,), lambda i: (i,)),
            pl.BlockSpec(memory_space=pltpu.SMEM),
            pl.BlockSpec(memory_space=pltpu.SMEM),
            pl.BlockSpec(memory_space=pltpu.SMEM),
            pl.BlockSpec(memory_space=pltpu.SMEM),
            pl.BlockSpec((NA * 4 // 128, 128), lambda i: (0, 0)),
            pl.BlockSpec((NA * 4 // 128, 128), lambda i: (0, 0)),
        ],
        out_specs=pl.BlockSpec(memory_space=pltpu.SMEM),
        out_shape=jax.ShapeDtypeStruct((3,), jnp.float32),
        scratch_shapes=[pltpu.SMEM((4,), jnp.float32)],
    )(sv, g0, g1, g2, w1t, b1c, w2, cell, v4, m4)


def kernel(q, v, nbr, offsets, mass, cell, W1, b1, W2, b2):
    qpad = jnp.pad(q, ((0, 0), (0, 1)))
    cellb = jnp.broadcast_to(cell[:, None], (3, 16))

    sv, g0, g1, g2 = _sc_stage(qpad, nbr.T, offsets.T, cellb)

    w1f = W1.reshape(HID)
    w2f = W2.reshape(HID)
    v4 = jnp.pad(v, ((0, 0), (0, 1))).reshape(NA * 4 // 128, 128)
    m4 = jnp.broadcast_to(mass[:, None], (NA, 4)).reshape(NA * 4 // 128, 128)
    return _tc_stage(sv, g0, g1, g2, w1f, b1, w2f, cell, v4, m4)
